# Initial kernel scaffold; baseline (speedup 1.0000x reference)
import jax
import jax.numpy as jnp
from jax.experimental import pallas as pl


def kernel(X, emb1, emb2, W, b):
    raise NotImplementedError("write your pallas kernel here")


# speedup vs baseline: 1.2477x; 1.2477x over previous
import functools

import jax
import jax.numpy as jnp
from jax import lax
from jax.experimental import pallas as pl
from jax.experimental.pallas import tpu as pltpu
from jax.experimental.pallas import tpu_sc as plsc

B, F, V, D = 16384, 26, 1040000, 16
NW = 32
ROWS_W = B // NW
GROUP_ROWS = 4
GROUP_IDX = GROUP_ROWS * F
CHUNK_ROWS = 16
GROUPS_PER_CHUNK = CHUNK_ROWS // GROUP_ROWS
CHUNK_IDX = CHUNK_ROWS * F
NCHUNKS = ROWS_W // CHUNK_ROWS
NGROUPS_W = ROWS_W // GROUP_ROWS

_mesh = plsc.VectorSubcoreMesh(core_axis_name="c", subcore_axis_name="s")


@functools.partial(
    pl.kernel,
    mesh=_mesh,
    compiler_params=pltpu.CompilerParams(
        needs_layout_passes=False, use_tc_tiling_on_sc=False),
    out_type=jax.ShapeDtypeStruct((B,), jnp.float32),
    scratch_types=[
        pltpu.VMEM((NGROUPS_W, GROUP_IDX), jnp.int32),
        pltpu.VMEM((NCHUNKS, GROUPS_PER_CHUNK, GROUP_IDX), jnp.int32),
        pltpu.VMEM((CHUNK_IDX, D), jnp.float32),
        pltpu.VMEM((CHUNK_IDX,), jnp.float32),
        pltpu.VMEM((ROWS_W,), jnp.float32),
        pltpu.VMEM((16,), jnp.float32),
        pltpu.VMEM((16,), jnp.float32),
        pltpu.SemaphoreType.DMA,
    ],
)
def _fm_kernel(x_hbm, xt_hbm, w_hbm, b_hbm, emb1_hbm, emb2_hbm, out_hbm,
               xv, xtv, e2b, e1b, outb, wv, bv, sem):
    wid = lax.axis_index("s") * 2 + lax.axis_index("c")
    pltpu.sync_copy(x_hbm.at[wid], xv)
    pltpu.sync_copy(xt_hbm.at[wid], xtv)
    pltpu.sync_copy(w_hbm, wv)
    pltpu.sync_copy(b_hbm, bv)
    w = wv[...]
    bias = bv[...]
    lanes = lax.iota(jnp.int32, 16)

    def chunk_body(c, carry):
        cps = []
        for g in range(GROUPS_PER_CHUNK):
            gg = c * GROUPS_PER_CHUNK + g
            dst2 = e2b.at[pl.ds(g * GROUP_IDX, GROUP_IDX)]
            dst1 = e1b.at[pl.ds(g * GROUP_IDX, GROUP_IDX)]
            cps.append(pltpu.async_copy(emb2_hbm.at[xv.at[gg]], dst2, sem))
            cps.append(pltpu.async_copy(emb1_hbm.at[xtv.at[c, g]], dst1, sem))
        for cp in cps:
            cp.wait()

        sec = jnp.zeros((16,), jnp.float32)
        for r in range(CHUNK_ROWS):
            s = jnp.zeros((16,), jnp.float32)
            q = jnp.zeros((16,), jnp.float32)
            for f in range(F):
                v = e2b[r * F + f]
                s = s + v
                q = q + v * v
            sec = jnp.where(lanes == r, jnp.sum(s * s - q), sec)

        first = jnp.zeros((16,), jnp.float32)
        for f in range(F):
            first = first + e1b[pl.ds(f * CHUNK_ROWS, CHUNK_ROWS)]

        tot = (first + 0.5 * sec) * w + bias
        outb[pl.ds(c * CHUNK_ROWS, CHUNK_ROWS)] = 1.0 / (1.0 + jnp.exp(-tot))
        return carry

    lax.fori_loop(0, NCHUNKS, chunk_body, 0)
    pltpu.sync_copy(outb, out_hbm.at[pl.ds(wid * ROWS_W, ROWS_W)])


def kernel(X, emb1, emb2, W, b):
    xi = X.astype(jnp.int32)
    xr = xi.reshape(NW, NGROUPS_W, GROUP_IDX)
    xt = (xi.reshape(NW, NCHUNKS, CHUNK_ROWS, F)
          .transpose(0, 1, 3, 2)
          .reshape(NW, NCHUNKS, GROUPS_PER_CHUNK, GROUP_IDX))
    wvec = jnp.broadcast_to(W.astype(jnp.float32).reshape(1), (16,))
    bvec = jnp.broadcast_to(b.astype(jnp.float32).reshape(1), (16,))
    out = _fm_kernel(xr, xt, wvec, bvec,
                     emb1.astype(jnp.float32).reshape(V),
                     emb2.astype(jnp.float32))
    return out.reshape(B, 1)


# speedup vs baseline: 1.2518x; 1.0033x over previous
import functools

import jax
import jax.numpy as jnp
from jax import lax
from jax.experimental import pallas as pl
from jax.experimental.pallas import tpu as pltpu
from jax.experimental.pallas import tpu_sc as plsc

B, F, V, D = 16384, 26, 1040000, 16
NW = 32
ROWS_W = B // NW
IDX_W = ROWS_W * F
CHUNK_ROWS = 16
CHUNK_IDX = CHUNK_ROWS * F
NCHUNKS = ROWS_W // CHUNK_ROWS

_mesh = plsc.VectorSubcoreMesh(core_axis_name="c", subcore_axis_name="s")


@functools.partial(
    pl.kernel,
    mesh=_mesh,
    compiler_params=pltpu.CompilerParams(
        needs_layout_passes=False, use_tc_tiling_on_sc=False),
    out_type=jax.ShapeDtypeStruct((B,), jnp.float32),
    scratch_types=[
        pltpu.VMEM((NCHUNKS, CHUNK_IDX), jnp.int32),
        pltpu.VMEM((IDX_W,), jnp.int32),
        pltpu.VMEM((CHUNK_IDX, D), jnp.float32),
        pltpu.VMEM((IDX_W,), jnp.float32),
        pltpu.VMEM((ROWS_W,), jnp.float32),
        pltpu.VMEM((16,), jnp.float32),
        pltpu.VMEM((16,), jnp.float32),
        pltpu.SemaphoreType.DMA,
        pltpu.SemaphoreType.DMA,
    ],
)
def _fm_kernel(x_hbm, xt_hbm, w_hbm, b_hbm, emb1_hbm, emb2_hbm, out_hbm,
               xv, xtv, e2b, e1all, outb, wv, bv, sem, sem1):
    wid = lax.axis_index("s") * 2 + lax.axis_index("c")
    pltpu.sync_copy(xt_hbm.at[wid], xtv)
    cp1 = pltpu.async_copy(emb1_hbm.at[xtv], e1all, sem1)
    pltpu.sync_copy(x_hbm.at[wid], xv)
    pltpu.sync_copy(w_hbm, wv)
    pltpu.sync_copy(b_hbm, bv)
    w = wv[...]
    bias = bv[...]
    lanes = lax.iota(jnp.int32, 16)
    cp1.wait()

    def chunk_body(c, carry):
        pltpu.async_copy(emb2_hbm.at[xv.at[c]], e2b, sem).wait()

        sec = jnp.zeros((16,), jnp.float32)
        for r in range(CHUNK_ROWS):
            s = jnp.zeros((16,), jnp.float32)
            q = jnp.zeros((16,), jnp.float32)
            for f in range(F):
                v = e2b[r * F + f]
                s = s + v
                q = q + v * v
            sec = jnp.where(lanes == r, jnp.sum(s * s - q), sec)

        first = jnp.zeros((16,), jnp.float32)
        for f in range(F):
            first = first + e1all[pl.ds(c * CHUNK_IDX + f * CHUNK_ROWS,
                                        CHUNK_ROWS)]

        tot = (first + 0.5 * sec) * w + bias
        outb[pl.ds(c * CHUNK_ROWS, CHUNK_ROWS)] = 1.0 / (1.0 + jnp.exp(-tot))
        return carry

    lax.fori_loop(0, NCHUNKS, chunk_body, 0)
    pltpu.sync_copy(outb, out_hbm.at[pl.ds(wid * ROWS_W, ROWS_W)])


def kernel(X, emb1, emb2, W, b):
    xi = X.astype(jnp.int32)
    xr = xi.reshape(NW, NCHUNKS, CHUNK_IDX)
    xt = (xi.reshape(NW, NCHUNKS, CHUNK_ROWS, F)
          .transpose(0, 1, 3, 2)
          .reshape(NW, IDX_W))
    wvec = jnp.broadcast_to(W.astype(jnp.float32).reshape(1), (16,))
    bvec = jnp.broadcast_to(b.astype(jnp.float32).reshape(1), (16,))
    out = _fm_kernel(xr, xt, wvec, bvec,
                     emb1.astype(jnp.float32).reshape(V),
                     emb2.astype(jnp.float32))
    return out.reshape(B, 1)
